# Initial kernel scaffold; baseline (speedup 1.0000x reference)
import jax
import jax.numpy as jnp
from jax.experimental import pallas as pl


def kernel(edge, sizes, Wr_in, br_in, Wr_out, br_out, Wu_in, bu_in, Wu_out, bu_out):
    raise NotImplementedError("write your pallas kernel here")


# speedup vs baseline: 1.7111x; 1.7111x over previous
import functools

import jax
import jax.numpy as jnp
from jax.experimental import pallas as pl
from jax.experimental.pallas import tpu as pltpu

EMB = 128
D_IN = 2 * EMB
SMOOTH = 12.0
N_NODES = 10000


def _mish(x):
    return x * jnp.tanh(jax.nn.softplus(x))


def _l1_body(ce_ref, co_ref, br_in_ref, Wr_out_ref, br_out_ref,
             Wu_in_ref, bu_in_ref, Wu_out_ref, bu_out_ref, out_ref):
    m_full = _mish(br_in_ref[...]) @ Wr_out_ref[...] + br_out_ref[...]
    m0 = m_full[:, :EMB]
    m1 = m_full[:, EMB:]
    ce = ce_ref[...].reshape(-1, 1)
    co = co_ref[...].reshape(-1, 1)
    has_e = ce > 0.0
    has_o = co > 0.0
    mx = jnp.maximum(m0, m1)
    smax = jnp.where(has_e & has_o, mx,
                     jnp.where(has_e, m0, jnp.where(has_o, m1, 0.0)))
    ssum = 1e-16 + ce * jnp.exp(SMOOTH * (m0 - smax)) \
                 + co * jnp.exp(SMOOTH * (m1 - smax))
    max_msg = jnp.log(ssum) / SMOOTH + smax
    h = _mish(max_msg @ Wu_in_ref[...][:EMB, :] + bu_in_ref[...])
    out_ref[...] = h @ Wu_out_ref[...] + bu_out_ref[...]


def _l1_apply(ce, co, br_in, Wr_out, br_out, Wu_in, bu_in, Wu_out, bu_out):
    B = 1000
    grid = (N_NODES // B,)
    full = lambda *s: pl.BlockSpec(s, lambda i: tuple(0 for _ in s))
    return pl.pallas_call(
        _l1_body,
        grid=grid,
        in_specs=[
            pl.BlockSpec((1, 1, B), lambda i: (i, 0, 0)),
            pl.BlockSpec((1, 1, B), lambda i: (i, 0, 0)),
            full(1, D_IN), full(D_IN, D_IN), full(1, D_IN),
            full(D_IN, D_IN), full(1, D_IN), full(D_IN, EMB), full(1, EMB),
        ],
        out_specs=pl.BlockSpec((B, EMB), lambda i: (i, 0)),
        out_shape=jax.ShapeDtypeStruct((N_NODES, EMB), jnp.float32),
    )(ce.reshape(-1, 1, B), co.reshape(-1, 1, B),
      br_in.reshape(1, -1), Wr_out, br_out.reshape(1, -1),
      Wu_in, bu_in.reshape(1, -1), Wu_out, bu_out.reshape(1, -1))


def _mlp_body(inp_ref, W1_ref, b1_ref, W2_ref, b2_ref, out_ref):
    x = inp_ref[...]
    h = _mish(x @ W1_ref[...] + b1_ref[...])
    out_ref[...] = x + (h @ W2_ref[...] + b2_ref[...])


def _mlp_apply(inp, W1, b1, W2, b2):
    E, D = inp.shape
    B = 640
    grid = (E // B,)
    full = lambda *s: pl.BlockSpec(s, lambda i: tuple(0 for _ in s))
    return pl.pallas_call(
        _mlp_body,
        grid=grid,
        in_specs=[
            pl.BlockSpec((B, D), lambda i: (i, 0)),
            full(D, D), full(1, D), full(D, D), full(1, D),
        ],
        out_specs=pl.BlockSpec((B, D), lambda i: (i, 0)),
        out_shape=jax.ShapeDtypeStruct((E, D), jnp.float32),
    )(inp, W1, b1.reshape(1, -1), W2, b2.reshape(1, -1))


def _upd_body(smax_ref, ssum_ref, obj_ref, Wu_in_ref, bu_in_ref,
              Wu_out_ref, bu_out_ref, out_ref):
    obj = obj_ref[...]
    max_msg = jnp.log(ssum_ref[...]) / SMOOTH + smax_ref[...]
    Wu = Wu_in_ref[...]
    pre = max_msg @ Wu[:EMB, :] + obj @ Wu[EMB:, :] + bu_in_ref[...]
    h = _mish(pre)
    out_ref[...] = obj + (h @ Wu_out_ref[...] + bu_out_ref[...])


def _upd_apply(smax, ssum, obj, Wu_in, bu_in, Wu_out, bu_out):
    B = 1000
    grid = (N_NODES // B,)
    full = lambda *s: pl.BlockSpec(s, lambda i: tuple(0 for _ in s))
    return pl.pallas_call(
        _upd_body,
        grid=grid,
        in_specs=[
            pl.BlockSpec((B, EMB), lambda i: (i, 0)),
            pl.BlockSpec((B, EMB), lambda i: (i, 0)),
            pl.BlockSpec((B, EMB), lambda i: (i, 0)),
            full(D_IN, D_IN), full(1, D_IN), full(D_IN, EMB), full(1, EMB),
        ],
        out_specs=pl.BlockSpec((B, EMB), lambda i: (i, 0)),
        out_shape=jax.ShapeDtypeStruct((N_NODES, EMB), jnp.float32),
    )(smax, ssum, obj, Wu_in, bu_in.reshape(1, -1), Wu_out, bu_out.reshape(1, -1))


def kernel(edge, sizes, Wr_in, br_in, Wr_out, br_out, Wu_in, bu_in, Wu_out, bu_out):
    N = sizes.shape[0]

    ce = jnp.zeros((N,), jnp.float32).at[edge[0::2]].add(1.0)
    co = jnp.zeros((N,), jnp.float32).at[edge[1::2]].add(1.0)
    obj = _l1_apply(ce, co, br_in, Wr_out, br_out, Wu_in, bu_in, Wu_out, bu_out)

    inp = jnp.take(obj, edge, axis=0).reshape(-1, D_IN)
    msg = _mlp_apply(inp, Wr_in, br_in, Wr_out, br_out)
    msgs = msg.reshape(-1, EMB)

    smax = jnp.full((N, EMB), -jnp.inf).at[edge].max(msgs)
    smax = jnp.where(jnp.isfinite(smax), smax, 0.0)
    exps = jnp.exp(SMOOTH * (msgs - jnp.take(smax, edge, axis=0)))
    ssum = 1e-16 + jnp.zeros((N, EMB), jnp.float32).at[edge].add(exps)

    return _upd_apply(smax, ssum, obj, Wu_in, bu_in, Wu_out, bu_out)


# speedup vs baseline: 2.5043x; 1.4636x over previous
import functools

import jax
import jax.numpy as jnp
from jax import lax
from jax.experimental import pallas as pl
from jax.experimental.pallas import tpu as pltpu
from jax.experimental.pallas import tpu_sc as plsc

EMB = 128
D_IN = 2 * EMB
SMOOTH = 12.0
N_NODES = 10000

NC, NS, L = 2, 16, 16
NW = NC * NS
NN = 320
N_PAD = NW * NN
NEG = -3.0e38


def _mish(x):
    return x * jnp.tanh(jax.nn.softplus(x))


def _wid():
    return lax.axis_index("s") * NC + lax.axis_index("c")


def _sc_hist(edge):
    E2 = edge.shape[0]
    PER = E2 // NW
    CH = 2000
    mesh = plsc.VectorSubcoreMesh(core_axis_name="c", subcore_axis_name="s")

    @functools.partial(
        pl.kernel, mesh=mesh,
        compiler_params=pltpu.CompilerParams(needs_layout_passes=False),
        out_type=jax.ShapeDtypeStruct((2, NW, N_PAD), jnp.float32),
        scratch_types=[
            pltpu.VMEM((CH,), jnp.int32),
            pltpu.VMEM((N_PAD,), jnp.float32),
            pltpu.VMEM((N_PAD,), jnp.float32),
        ],
    )
    def k(edge_hbm, out_hbm, buf, ce_t, co_t):
        wid = _wid()
        base = wid * PER

        def zero(i, _):
            ce_t[pl.ds(i * L, L)] = jnp.zeros((L,), jnp.float32)
            co_t[pl.ds(i * L, L)] = jnp.zeros((L,), jnp.float32)
            return ()

        lax.fori_loop(0, N_PAD // L, zero, ())

        ones = jnp.ones((L,), jnp.float32)
        even = (lax.iota(jnp.int32, L) % 2) == 0
        odd = jnp.logical_not(even)

        def chunk(i, _):
            pltpu.sync_copy(edge_hbm.at[pl.ds(base + i * CH, CH)], buf)

            def inner(j, _):
                v = buf[pl.ds(j * L, L)]
                plsc.addupdate_scatter(ce_t, [v], ones, mask=even)
                plsc.addupdate_scatter(co_t, [v], ones, mask=odd)
                return ()

            lax.fori_loop(0, CH // L, inner, ())
            return ()

        lax.fori_loop(0, PER // CH, chunk, ())
        pltpu.sync_copy(ce_t, out_hbm.at[0, wid])
        pltpu.sync_copy(co_t, out_hbm.at[1, wid])

    return k(edge)


def _sc_gather(obj, idx):
    E2 = idx.shape[0]
    PER = E2 // NW
    C = 400
    mesh = plsc.VectorSubcoreMesh(core_axis_name="c", subcore_axis_name="s")

    @functools.partial(
        pl.kernel, mesh=mesh,
        compiler_params=pltpu.CompilerParams(needs_layout_passes=False),
        out_type=jax.ShapeDtypeStruct((E2, EMB), jnp.float32),
        scratch_types=[
            pltpu.VMEM((C,), jnp.int32),
            pltpu.VMEM((C, EMB), jnp.float32),
            pltpu.SemaphoreType.DMA,
        ],
    )
    def k(obj_hbm, idx_hbm, out_hbm, idx_v, rows_v, sem):
        wid = _wid()
        base = wid * PER

        def chunk(i, _):
            off = base + i * C
            pltpu.sync_copy(idx_hbm.at[pl.ds(off, C)], idx_v)
            pltpu.async_copy(obj_hbm.at[idx_v], rows_v, sem).wait()
            pltpu.sync_copy(rows_v, out_hbm.at[pl.ds(off, C)])
            return ()

        lax.fori_loop(0, PER // C, chunk, ())

    return k(obj, idx)


def _sc_seglse(msgs, perm_pad, dst_pad, starts_pad):
    C = 256
    mesh = plsc.VectorSubcoreMesh(core_axis_name="c", subcore_axis_name="s")

    @functools.partial(
        pl.kernel, mesh=mesh,
        compiler_params=pltpu.CompilerParams(needs_layout_passes=False),
        out_type=(jax.ShapeDtypeStruct((N_PAD, EMB), jnp.float32),
                  jax.ShapeDtypeStruct((N_PAD, EMB), jnp.float32)),
        scratch_types=[
            pltpu.VMEM((C,), jnp.int32),
            pltpu.VMEM((C,), jnp.int32),
            pltpu.VMEM((C, EMB), jnp.float32),
            pltpu.VMEM((NN, EMB), jnp.float32),
            pltpu.VMEM((NN, EMB), jnp.float32),
            pltpu.VMEM((2 * L,), jnp.int32),
            pltpu.SemaphoreType.DMA,
        ],
    )
    def k(msgs_hbm, perm_hbm, dst_hbm, starts_hbm, smax_hbm, ssum_hbm,
          idx_v, dst_v, rows_v, tmax, tsum, bnd_v, sem):
        wid = _wid()
        n0 = wid * NN

        pltpu.sync_copy(starts_hbm.at[pl.ds(n0, L)], bnd_v.at[pl.ds(0, L)])
        pltpu.sync_copy(starts_hbm.at[pl.ds(n0 + NN, L)], bnd_v.at[pl.ds(L, L)])

        def init(i, _):
            for d in range(EMB // L):
                sl = pl.ds(d * L, L)
                tmax[i, sl] = jnp.full((L,), NEG, jnp.float32)
                tsum[i, sl] = jnp.zeros((L,), jnp.float32)
            return ()

        lax.fori_loop(0, NN, init, ())

        r0 = bnd_v[pl.ds(0, L)][0]
        r1 = bnd_v[pl.ds(L, L)][0]
        rc0 = r0 - lax.rem(r0, 8)
        nchunks = lax.div(r1 - rc0 + C - 1, C)

        def make_pass(update_row):
            def chunk(kk, _):
                rc = pl.multiple_of(rc0 + kk * C, 8)
                pltpu.sync_copy(perm_hbm.at[pl.ds(rc, C)], idx_v)
                pltpu.sync_copy(dst_hbm.at[pl.ds(rc, C)], dst_v)
                pltpu.async_copy(msgs_hbm.at[idx_v], rows_v, sem).wait()

                def grp(g, _):
                    dvec = dst_v[pl.ds(g * L, L)] - n0
                    for jj in range(L):
                        p = dvec[jj]
                        j = g * L + jj
                        ok = jnp.logical_and(p >= 0, p < NN)
                        lax.cond(ok, lambda: update_row(j, p), lambda: None)
                    return ()

                lax.fori_loop(0, C // L, grp, ())
                return ()

            return chunk

        def upd_max(j, p):
            for d in range(EMB // L):
                sl = pl.ds(d * L, L)
                tmax[p, sl] = jnp.maximum(tmax[p, sl], rows_v[j, sl])

        def upd_sum(j, p):
            for d in range(EMB // L):
                sl = pl.ds(d * L, L)
                tsum[p, sl] = tsum[p, sl] + jnp.exp(
                    SMOOTH * (rows_v[j, sl] - tmax[p, sl]))

        lax.fori_loop(0, nchunks, make_pass(upd_max), ())
        lax.fori_loop(0, nchunks, make_pass(upd_sum), ())

        pltpu.sync_copy(tmax, smax_hbm.at[pl.ds(n0, NN)])
        pltpu.sync_copy(tsum, ssum_hbm.at[pl.ds(n0, NN)])

    return k(msgs, perm_pad, dst_pad, starts_pad)


def _l1_body(cnt_ref, br_in_ref, Wr_out_ref, br_out_ref,
             Wu_in_ref, bu_in_ref, Wu_out_ref, bu_out_ref, out_ref):
    m_full = _mish(br_in_ref[...]) @ Wr_out_ref[...] + br_out_ref[...]
    m0 = m_full[:, :EMB]
    m1 = m_full[:, EMB:]
    cnt = cnt_ref[...]
    ce = jnp.sum(cnt[0], axis=0).reshape(-1, 1)
    co = jnp.sum(cnt[1], axis=0).reshape(-1, 1)
    has_e = ce > 0.0
    has_o = co > 0.0
    mx = jnp.maximum(m0, m1)
    smax = jnp.where(has_e & has_o, mx,
                     jnp.where(has_e, m0, jnp.where(has_o, m1, 0.0)))
    ssum = 1e-16 + ce * jnp.exp(SMOOTH * (m0 - smax)) \
                 + co * jnp.exp(SMOOTH * (m1 - smax))
    max_msg = jnp.log(ssum) / SMOOTH + smax
    h = _mish(max_msg @ Wu_in_ref[...][:EMB, :] + bu_in_ref[...])
    out_ref[...] = h @ Wu_out_ref[...] + bu_out_ref[...]


def _l1_apply(counts, br_in, Wr_out, br_out, Wu_in, bu_in, Wu_out, bu_out):
    B = 1280
    grid = (N_PAD // B,)
    full = lambda *s: pl.BlockSpec(s, lambda i: tuple(0 for _ in s))
    return pl.pallas_call(
        _l1_body,
        grid=grid,
        in_specs=[
            pl.BlockSpec((2, NW, B), lambda i: (0, 0, i)),
            full(1, D_IN), full(D_IN, D_IN), full(1, D_IN),
            full(D_IN, D_IN), full(1, D_IN), full(D_IN, EMB), full(1, EMB),
        ],
        out_specs=pl.BlockSpec((B, EMB), lambda i: (i, 0)),
        out_shape=jax.ShapeDtypeStruct((N_PAD, EMB), jnp.float32),
    )(counts, br_in.reshape(1, -1), Wr_out, br_out.reshape(1, -1),
      Wu_in, bu_in.reshape(1, -1), Wu_out, bu_out.reshape(1, -1))


def _mlp_body(inp_ref, W1_ref, b1_ref, W2_ref, b2_ref, out_ref):
    x = inp_ref[...]
    h = _mish(x @ W1_ref[...] + b1_ref[...])
    out_ref[...] = x + (h @ W2_ref[...] + b2_ref[...])


def _mlp_apply(inp, W1, b1, W2, b2):
    E, D = inp.shape
    B = 640
    grid = (E // B,)
    full = lambda *s: pl.BlockSpec(s, lambda i: tuple(0 for _ in s))
    return pl.pallas_call(
        _mlp_body,
        grid=grid,
        in_specs=[
            pl.BlockSpec((B, D), lambda i: (i, 0)),
            full(D, D), full(1, D), full(D, D), full(1, D),
        ],
        out_specs=pl.BlockSpec((B, D), lambda i: (i, 0)),
        out_shape=jax.ShapeDtypeStruct((E, D), jnp.float32),
    )(inp, W1, b1.reshape(1, -1), W2, b2.reshape(1, -1))


def _upd_body(smax_ref, ssum_ref, obj_ref, Wu_in_ref, bu_in_ref,
              Wu_out_ref, bu_out_ref, out_ref):
    obj = obj_ref[...]
    smax = smax_ref[...]
    smax = jnp.where(smax > -1e30, smax, 0.0)
    max_msg = jnp.log(ssum_ref[...] + 1e-16) / SMOOTH + smax
    Wu = Wu_in_ref[...]
    pre = max_msg @ Wu[:EMB, :] + obj @ Wu[EMB:, :] + bu_in_ref[...]
    h = _mish(pre)
    out_ref[...] = obj + (h @ Wu_out_ref[...] + bu_out_ref[...])


def _upd_apply(smax, ssum, obj, Wu_in, bu_in, Wu_out, bu_out):
    B = 1000
    grid = (N_NODES // B,)
    full = lambda *s: pl.BlockSpec(s, lambda i: tuple(0 for _ in s))
    return pl.pallas_call(
        _upd_body,
        grid=grid,
        in_specs=[
            pl.BlockSpec((B, EMB), lambda i: (i, 0)),
            pl.BlockSpec((B, EMB), lambda i: (i, 0)),
            pl.BlockSpec((B, EMB), lambda i: (i, 0)),
            full(D_IN, D_IN), full(1, D_IN), full(D_IN, EMB), full(1, EMB),
        ],
        out_specs=pl.BlockSpec((B, EMB), lambda i: (i, 0)),
        out_shape=jax.ShapeDtypeStruct((N_NODES, EMB), jnp.float32),
    )(smax, ssum, obj, Wu_in, bu_in.reshape(1, -1), Wu_out, bu_out.reshape(1, -1))


def kernel(edge, sizes, Wr_in, br_in, Wr_out, br_out, Wu_in, bu_in, Wu_out, bu_out):
    N = sizes.shape[0]
    E2 = edge.shape[0]

    perm = jnp.argsort(edge).astype(jnp.int32)
    perm_pad = jnp.concatenate([perm, jnp.zeros((320,), jnp.int32)])
    dst_pad = jnp.concatenate(
        [jnp.take(edge, perm), jnp.full((320,), -1, jnp.int32)])

    counts = _sc_hist(edge)
    deg = jnp.sum(counts, axis=(0, 1)).astype(jnp.int32)
    starts = jnp.concatenate([jnp.zeros((1,), jnp.int32), jnp.cumsum(deg)])
    starts_pad = jnp.concatenate(
        [starts, jnp.full((2 * L - 1,), E2, jnp.int32)]).astype(jnp.int32)

    obj = _l1_apply(counts, br_in, Wr_out, br_out,
                    Wu_in, bu_in, Wu_out, bu_out)[:N]

    inp = _sc_gather(obj, edge).reshape(-1, D_IN)
    msg = _mlp_apply(inp, Wr_in, br_in, Wr_out, br_out)
    msgs = msg.reshape(-1, EMB)

    smax_p, ssum_p = _sc_seglse(msgs, perm_pad, dst_pad, starts_pad)
    return _upd_apply(smax_p[:N], ssum_p[:N], obj,
                      Wu_in, bu_in, Wu_out, bu_out)


# speedup vs baseline: 4.5198x; 1.8048x over previous
import functools

import jax
import jax.numpy as jnp
from jax import lax
from jax.experimental import pallas as pl
from jax.experimental.pallas import tpu as pltpu
from jax.experimental.pallas import tpu_sc as plsc

EMB = 128
D_IN = 2 * EMB
SMOOTH = 12.0
N_NODES = 10000

NC, NS, L = 2, 16, 16
NW = NC * NS
NN = 320
N_PAD = NW * NN
NEG = -3.0e38


def _mish(x):
    return x * jnp.tanh(jax.nn.softplus(x))


def _wid():
    return lax.axis_index("s") * NC + lax.axis_index("c")


def _sc_hist(edge):
    E2 = edge.shape[0]
    PER = E2 // NW
    CH = 2000
    mesh = plsc.VectorSubcoreMesh(core_axis_name="c", subcore_axis_name="s")

    @functools.partial(
        pl.kernel, mesh=mesh,
        compiler_params=pltpu.CompilerParams(needs_layout_passes=False),
        out_type=jax.ShapeDtypeStruct((2, NW, N_PAD), jnp.float32),
        scratch_types=[
            pltpu.VMEM((CH,), jnp.int32),
            pltpu.VMEM((N_PAD,), jnp.float32),
            pltpu.VMEM((N_PAD,), jnp.float32),
        ],
    )
    def k(edge_hbm, out_hbm, buf, ce_t, co_t):
        wid = _wid()
        base = wid * PER

        def zero(i, _):
            ce_t[pl.ds(i * L, L)] = jnp.zeros((L,), jnp.float32)
            co_t[pl.ds(i * L, L)] = jnp.zeros((L,), jnp.float32)
            return ()

        lax.fori_loop(0, N_PAD // L, zero, ())

        ones = jnp.ones((L,), jnp.float32)
        even = (lax.iota(jnp.int32, L) % 2) == 0
        odd = jnp.logical_not(even)

        def chunk(i, _):
            pltpu.sync_copy(edge_hbm.at[pl.ds(base + i * CH, CH)], buf)

            def inner(j, _):
                v = buf[pl.ds(j * L, L)]
                plsc.addupdate_scatter(ce_t, [v], ones, mask=even)
                plsc.addupdate_scatter(co_t, [v], ones, mask=odd)
                return ()

            lax.fori_loop(0, CH // L, inner, ())
            return ()

        lax.fori_loop(0, PER // CH, chunk, ())
        pltpu.sync_copy(ce_t, out_hbm.at[0, wid])
        pltpu.sync_copy(co_t, out_hbm.at[1, wid])

    return k(edge)


def _sc_gather(obj, idx):
    E2 = idx.shape[0]
    PER = E2 // NW
    C = 400
    mesh = plsc.VectorSubcoreMesh(core_axis_name="c", subcore_axis_name="s")

    @functools.partial(
        pl.kernel, mesh=mesh,
        compiler_params=pltpu.CompilerParams(needs_layout_passes=False),
        out_type=jax.ShapeDtypeStruct((E2, EMB), jnp.float32),
        scratch_types=[
            pltpu.VMEM((C,), jnp.int32),
            pltpu.VMEM((C, EMB), jnp.float32),
            pltpu.SemaphoreType.DMA,
        ],
    )
    def k(obj_hbm, idx_hbm, out_hbm, idx_v, rows_v, sem):
        wid = _wid()
        base = wid * PER

        def chunk(i, _):
            off = base + i * C
            pltpu.sync_copy(idx_hbm.at[pl.ds(off, C)], idx_v)
            pltpu.async_copy(obj_hbm.at[idx_v], rows_v, sem).wait()
            pltpu.sync_copy(rows_v, out_hbm.at[pl.ds(off, C)])
            return ()

        lax.fori_loop(0, PER // C, chunk, ())

    return k(obj, idx)


def _sc_seglse(msgs, perm_pad, dst_pad, starts_pad):
    C = 128
    mesh = plsc.VectorSubcoreMesh(core_axis_name="c", subcore_axis_name="s")

    @functools.partial(
        pl.kernel, mesh=mesh,
        compiler_params=pltpu.CompilerParams(needs_layout_passes=False),
        out_type=(jax.ShapeDtypeStruct((N_PAD, EMB), jnp.float32),
                  jax.ShapeDtypeStruct((N_PAD, EMB), jnp.float32)),
        scratch_types=[
            pltpu.VMEM((C,), jnp.int32),
            pltpu.VMEM((2, C), jnp.int32),
            pltpu.VMEM((2, C, EMB), jnp.float32),
            pltpu.VMEM((NN + 1, EMB), jnp.float32),
            pltpu.VMEM((NN + 1, EMB), jnp.float32),
            pltpu.VMEM((2 * L,), jnp.int32),
            pltpu.SemaphoreType.DMA,
            pltpu.SemaphoreType.DMA,
        ],
    )
    def k(msgs_hbm, perm_hbm, dst_hbm, starts_hbm, smax_hbm, ssum_hbm,
          idx_v, dst_v, rows_v, tmax, tsum, bnd_v, sem0, sem1):
        wid = _wid()
        n0 = wid * NN

        pltpu.sync_copy(starts_hbm.at[pl.ds(n0, L)], bnd_v.at[pl.ds(0, L)])
        pltpu.sync_copy(starts_hbm.at[pl.ds(n0 + NN, L)], bnd_v.at[pl.ds(L, L)])

        def init(i, _):
            for d in range(EMB // L):
                sl = pl.ds(d * L, L)
                tmax[i, sl] = jnp.full((L,), NEG, jnp.float32)
                tsum[i, sl] = jnp.zeros((L,), jnp.float32)
            return ()

        lax.fori_loop(0, NN + 1, init, ())

        r0 = bnd_v[pl.ds(0, L)][0]
        r1 = bnd_v[pl.ds(L, L)][0]
        rc0 = r0 - lax.rem(r0, 8)
        nchunks = lax.div(r1 - rc0 + C - 1, C)

        sems = (sem0, sem1)
        ND = EMB // L
        NEGV = jnp.full((L,), NEG, jnp.float32)
        ZV = jnp.zeros((L,), jnp.float32)

        def issue(kk, b):
            rc = pl.multiple_of(rc0 + kk * C, 8)
            pltpu.sync_copy(perm_hbm.at[pl.ds(rc, C)], idx_v)
            pltpu.sync_copy(dst_hbm.at[pl.ds(rc, C)], dst_v.at[b])
            pltpu.async_copy(msgs_hbm.at[idx_v], rows_v.at[b], sems[b])

        def run_pass(row_step, carry0, final_flush):
            @pl.when(nchunks > 0)
            def _():
                issue(0, 0)

            def phase(kk, b, carry):
                def work(carry):
                    pltpu.make_async_copy(
                        msgs_hbm.at[idx_v], rows_v.at[b], sems[b]).wait()

                    @pl.when(kk + 1 < nchunks)
                    def _():
                        issue(kk + 1, 1 - b)

                    def grp(g, carry):
                        dvec = dst_v[b, pl.ds(g * L, L)] - n0
                        ok = jnp.logical_and(dvec >= 0, dvec < NN)
                        pc = jnp.where(ok, dvec, NN)
                        for jj in range(L):
                            carry = row_step(b, g * L + jj, pc[jj], carry)
                        return carry

                    return lax.fori_loop(0, C // L, grp, carry)

                return lax.cond(kk < nchunks, work, lambda c: c, carry)

            def pair(t, carry):
                carry = phase(2 * t, 0, carry)
                carry = phase(2 * t + 1, 1, carry)
                return carry

            carry = lax.fori_loop(0, lax.div(nchunks + 1, 2), pair, carry0)
            final_flush(carry)

        def step1(b, j, p, carry):
            pcur = carry[0]
            accs = carry[1:]

            def flush(accs):
                for d in range(ND):
                    tmax[pcur, pl.ds(d * L, L)] = accs[d]
                return (NEGV,) * ND

            accs = lax.cond(p != pcur, flush, lambda a: a, accs)
            accs = tuple(
                jnp.maximum(accs[d], rows_v[b, j, pl.ds(d * L, L)])
                for d in range(ND))
            return (p,) + accs

        def fin1(carry):
            pcur = carry[0]
            for d in range(ND):
                tmax[pcur, pl.ds(d * L, L)] = carry[1 + d]

        run_pass(step1, (NN,) + (NEGV,) * ND, fin1)

        def step2(b, j, p, carry):
            pcur = carry[0]
            accs = carry[1:1 + ND]
            mx = carry[1 + ND:]

            def flush(args):
                accs, _ = args
                for d in range(ND):
                    tsum[pcur, pl.ds(d * L, L)] = accs[d]
                newmx = tuple(tmax[p, pl.ds(d * L, L)] for d in range(ND))
                return (ZV,) * ND, newmx

            accs, mx = lax.cond(p != pcur, flush, lambda a: a, (accs, mx))
            accs = tuple(
                accs[d] + jnp.exp(SMOOTH * (rows_v[b, j, pl.ds(d * L, L)] - mx[d]))
                for d in range(ND))
            return (p,) + accs + mx

        def fin2(carry):
            pcur = carry[0]
            for d in range(ND):
                tsum[pcur, pl.ds(d * L, L)] = carry[1 + d]

        run_pass(step2, (NN,) + (ZV,) * ND + (ZV,) * ND, fin2)

        pltpu.sync_copy(tmax.at[pl.ds(0, NN)], smax_hbm.at[pl.ds(n0, NN)])
        pltpu.sync_copy(tsum.at[pl.ds(0, NN)], ssum_hbm.at[pl.ds(n0, NN)])

    return k(msgs, perm_pad, dst_pad, starts_pad)


def _l1_body(cnt_ref, br_in_ref, Wr_out_ref, br_out_ref,
             Wu_in_ref, bu_in_ref, Wu_out_ref, bu_out_ref, out_ref):
    m_full = _mish(br_in_ref[...]) @ Wr_out_ref[...] + br_out_ref[...]
    m0 = m_full[:, :EMB]
    m1 = m_full[:, EMB:]
    cnt = cnt_ref[...]
    ce = jnp.sum(cnt[0], axis=0).reshape(-1, 1)
    co = jnp.sum(cnt[1], axis=0).reshape(-1, 1)
    has_e = ce > 0.0
    has_o = co > 0.0
    mx = jnp.maximum(m0, m1)
    smax = jnp.where(has_e & has_o, mx,
                     jnp.where(has_e, m0, jnp.where(has_o, m1, 0.0)))
    ssum = 1e-16 + ce * jnp.exp(SMOOTH * (m0 - smax)) \
                 + co * jnp.exp(SMOOTH * (m1 - smax))
    max_msg = jnp.log(ssum) / SMOOTH + smax
    h = _mish(max_msg @ Wu_in_ref[...][:EMB, :] + bu_in_ref[...])
    out_ref[...] = h @ Wu_out_ref[...] + bu_out_ref[...]


def _l1_apply(counts, br_in, Wr_out, br_out, Wu_in, bu_in, Wu_out, bu_out):
    B = 1280
    grid = (N_PAD // B,)
    full = lambda *s: pl.BlockSpec(s, lambda i: tuple(0 for _ in s))
    return pl.pallas_call(
        _l1_body,
        grid=grid,
        in_specs=[
            pl.BlockSpec((2, NW, B), lambda i: (0, 0, i)),
            full(1, D_IN), full(D_IN, D_IN), full(1, D_IN),
            full(D_IN, D_IN), full(1, D_IN), full(D_IN, EMB), full(1, EMB),
        ],
        out_specs=pl.BlockSpec((B, EMB), lambda i: (i, 0)),
        out_shape=jax.ShapeDtypeStruct((N_PAD, EMB), jnp.float32),
    )(counts, br_in.reshape(1, -1), Wr_out, br_out.reshape(1, -1),
      Wu_in, bu_in.reshape(1, -1), Wu_out, bu_out.reshape(1, -1))


def _mlp_body(inp_ref, W1_ref, b1_ref, W2_ref, b2_ref, out_ref):
    x = inp_ref[...]
    h = _mish(x @ W1_ref[...] + b1_ref[...])
    out_ref[...] = x + (h @ W2_ref[...] + b2_ref[...])


def _mlp_apply(inp, W1, b1, W2, b2):
    E, D = inp.shape
    B = 640
    grid = (E // B,)
    full = lambda *s: pl.BlockSpec(s, lambda i: tuple(0 for _ in s))
    return pl.pallas_call(
        _mlp_body,
        grid=grid,
        in_specs=[
            pl.BlockSpec((B, D), lambda i: (i, 0)),
            full(D, D), full(1, D), full(D, D), full(1, D),
        ],
        out_specs=pl.BlockSpec((B, D), lambda i: (i, 0)),
        out_shape=jax.ShapeDtypeStruct((E, D), jnp.float32),
    )(inp, W1, b1.reshape(1, -1), W2, b2.reshape(1, -1))


def _upd_body(smax_ref, ssum_ref, obj_ref, Wu_in_ref, bu_in_ref,
              Wu_out_ref, bu_out_ref, out_ref):
    obj = obj_ref[...]
    smax = smax_ref[...]
    smax = jnp.where(smax > -1e30, smax, 0.0)
    max_msg = jnp.log(ssum_ref[...] + 1e-16) / SMOOTH + smax
    Wu = Wu_in_ref[...]
    pre = max_msg @ Wu[:EMB, :] + obj @ Wu[EMB:, :] + bu_in_ref[...]
    h = _mish(pre)
    out_ref[...] = obj + (h @ Wu_out_ref[...] + bu_out_ref[...])


def _upd_apply(smax, ssum, obj, Wu_in, bu_in, Wu_out, bu_out):
    B = 1000
    grid = (N_NODES // B,)
    full = lambda *s: pl.BlockSpec(s, lambda i: tuple(0 for _ in s))
    return pl.pallas_call(
        _upd_body,
        grid=grid,
        in_specs=[
            pl.BlockSpec((B, EMB), lambda i: (i, 0)),
            pl.BlockSpec((B, EMB), lambda i: (i, 0)),
            pl.BlockSpec((B, EMB), lambda i: (i, 0)),
            full(D_IN, D_IN), full(1, D_IN), full(D_IN, EMB), full(1, EMB),
        ],
        out_specs=pl.BlockSpec((B, EMB), lambda i: (i, 0)),
        out_shape=jax.ShapeDtypeStruct((N_NODES, EMB), jnp.float32),
    )(smax, ssum, obj, Wu_in, bu_in.reshape(1, -1), Wu_out, bu_out.reshape(1, -1))


def kernel(edge, sizes, Wr_in, br_in, Wr_out, br_out, Wu_in, bu_in, Wu_out, bu_out):
    N = sizes.shape[0]
    E2 = edge.shape[0]

    perm = jnp.argsort(edge).astype(jnp.int32)
    perm_pad = jnp.concatenate([perm, jnp.zeros((320,), jnp.int32)])
    dst_pad = jnp.concatenate(
        [jnp.take(edge, perm), jnp.full((320,), -1, jnp.int32)])

    counts = _sc_hist(edge)
    deg = jnp.sum(counts, axis=(0, 1)).astype(jnp.int32)
    starts = jnp.concatenate([jnp.zeros((1,), jnp.int32), jnp.cumsum(deg)])
    starts_pad = jnp.concatenate(
        [starts, jnp.full((2 * L - 1,), E2, jnp.int32)]).astype(jnp.int32)

    obj = _l1_apply(counts, br_in, Wr_out, br_out,
                    Wu_in, bu_in, Wu_out, bu_out)[:N]

    inp = _sc_gather(obj, edge).reshape(-1, D_IN)
    msg = _mlp_apply(inp, Wr_in, br_in, Wr_out, br_out)
    msgs = msg.reshape(-1, EMB)

    smax_p, ssum_p = _sc_seglse(msgs, perm_pad, dst_pad, starts_pad)
    return _upd_apply(smax_p[:N], ssum_p[:N], obj,
                      Wu_in, bu_in, Wu_out, bu_out)
